# baseline (speedup 1.0000x reference)
import functools

import jax
import jax.numpy as jnp
from jax import lax
from jax.experimental import pallas as pl
from jax.experimental.pallas import tpu as pltpu
from jax.experimental.pallas import tpu_sc as plsc

N = 10000
E = 320000
D = 128
BN_EPS = 1e-5

NC = 2
NS = 16
NW = NC * NS
EPT = E // NW
CK = 80
CH = EPT // CK
NP = 10112
RPT = NP // NS
NB = 1
NGP = CH // NB
NSL = 4
NFULL = RPT // CK
REM = RPT - NFULL * CK


def _sc_agg(h, srcg, dstg):
    mesh = plsc.VectorSubcoreMesh(core_axis_name="c", subcore_axis_name="s")

    @functools.partial(
        pl.kernel,
        mesh=mesh,
        out_type=jax.ShapeDtypeStruct((NC, NP, D), jnp.float32),
        scratch_types=[
            pltpu.VMEM((NSL, NB, CK), jnp.int32),
            pltpu.VMEM((NSL, NB, CK), jnp.int32),
            pltpu.VMEM((2, NB, CK, D), jnp.float32),
            pltpu.VMEM_SHARED((NP, D), jnp.float32),
            pltpu.SemaphoreType.DMA((2, NB)),
            pltpu.SemaphoreType.DMA((2, NB)),
            pltpu.SemaphoreType.DMA((NSL,)),
            pltpu.SemaphoreType.DMA((NSL,)),
        ],
    )
    def k(h_hbm, srcg_hbm, dstg_hbm, out_hbm,
          sidx, didx, rows_v, acc, gsem, ssem, isems, isemd):
        cid = lax.axis_index("c")
        sid = lax.axis_index("s")
        wid = cid * NS + sid
        r0 = sid * RPT

        pltpu.async_copy(srcg_hbm.at[wid, 0], sidx.at[0], isems.at[0])
        pltpu.async_copy(dstg_hbm.at[wid, 0], didx.at[0], isemd.at[0])
        pltpu.async_copy(srcg_hbm.at[wid, 1], sidx.at[1], isems.at[1])
        pltpu.async_copy(dstg_hbm.at[wid, 1], didx.at[1], isemd.at[1])

        zv = jnp.zeros((16,), jnp.float32)

        @pl.loop(0, CK)
        def _(i):
            for q in range(D // 16):
                rows_v[0, 0, i, pl.ds(q * 16, 16)] = zv

        @pl.loop(0, NFULL)
        def _(kk):
            pltpu.async_copy(rows_v.at[0, 0],
                             acc.at[pl.ds(r0 + kk * CK, CK)], gsem.at[0, 0])

        @pl.loop(0, NFULL)
        def _(kk):
            pltpu.make_async_copy(
                rows_v.at[0, 0], acc.at[pl.ds(r0 + kk * CK, CK)],
                gsem.at[0, 0]).wait()

        if REM:
            pltpu.sync_copy(rows_v.at[0, 0, pl.ds(0, REM)],
                            acc.at[pl.ds(r0 + NFULL * CK, REM)])

        pltpu.make_async_copy(srcg_hbm.at[wid, 0], sidx.at[0],
                              isems.at[0]).wait()
        pltpu.make_async_copy(dstg_hbm.at[wid, 0], didx.at[0],
                              isemd.at[0]).wait()
        plsc.subcore_barrier()
        for b in range(NB):
            pltpu.async_copy(h_hbm.at[sidx.at[0, b]], rows_v.at[0, b],
                             gsem.at[0, b])

        @pl.loop(0, NGP)
        def _(g):
            w = lax.rem(g, 2)
            s = lax.rem(g, NSL)

            @pl.when(g + 2 < NGP)
            def _():
                s2 = lax.rem(g + 2, NSL)
                pltpu.async_copy(srcg_hbm.at[wid, g + 2], sidx.at[s2],
                                 isems.at[s2])
                pltpu.async_copy(dstg_hbm.at[wid, g + 2], didx.at[s2],
                                 isemd.at[s2])

            for b in range(NB):
                pltpu.make_async_copy(
                    h_hbm.at[sidx.at[s, b]], rows_v.at[w, b], gsem.at[w, b]
                ).wait()
                pltpu.async_copy(
                    rows_v.at[w, b], acc.at[didx.at[s, b]], ssem.at[w, b],
                    add=True)

            @pl.when(g > 0)
            def _():
                wp = lax.rem(g + 1, 2)
                sp = lax.rem(g + NSL - 1, NSL)
                for b in range(NB):
                    pltpu.make_async_copy(
                        rows_v.at[wp, b], acc.at[didx.at[sp, b]],
                        ssem.at[wp, b]).wait()

            @pl.when(g + 1 < NGP)
            def _():
                wn = lax.rem(g + 1, 2)
                sn = lax.rem(g + 1, NSL)
                pltpu.make_async_copy(srcg_hbm.at[wid, g + 1], sidx.at[sn],
                                      isems.at[sn]).wait()
                pltpu.make_async_copy(dstg_hbm.at[wid, g + 1], didx.at[sn],
                                      isemd.at[sn]).wait()
                for b in range(NB):
                    pltpu.async_copy(h_hbm.at[sidx.at[sn, b]],
                                     rows_v.at[wn, b], gsem.at[wn, b])

        wl = (NGP - 1) % 2
        sl = (NGP - 1) % NSL
        for b in range(NB):
            pltpu.make_async_copy(rows_v.at[wl, b], acc.at[didx.at[sl, b]],
                                  ssem.at[wl, b]).wait()

        plsc.subcore_barrier()

        for q in range(2 * NB):
            pltpu.async_copy(acc.at[pl.ds(r0 + q * CK, CK)],
                             rows_v.at[q // NB, q % NB],
                             ssem.at[q // NB, q % NB])

        @pl.loop(0, NFULL)
        def _(kk):
            q = lax.rem(kk, 2 * NB)
            qw = q // NB
            qb = lax.rem(q, NB)
            pltpu.make_async_copy(
                acc.at[pl.ds(r0 + kk * CK, CK)], rows_v.at[qw, qb],
                ssem.at[qw, qb]).wait()
            pltpu.async_copy(rows_v.at[qw, qb],
                             out_hbm.at[cid, pl.ds(r0 + kk * CK, CK)],
                             gsem.at[qw, qb])

            @pl.when(kk + 2 * NB < NFULL)
            def _():
                pltpu.make_async_copy(
                    rows_v.at[qw, qb],
                    out_hbm.at[cid, pl.ds(r0 + kk * CK, CK)],
                    gsem.at[qw, qb]).wait()
                pltpu.async_copy(
                    acc.at[pl.ds(r0 + (kk + 2 * NB) * CK, CK)],
                    rows_v.at[qw, qb], ssem.at[qw, qb])

        for q in range(2 * NB):
            kk = NFULL - 2 * NB + q
            pltpu.make_async_copy(
                rows_v.at[(kk % (2 * NB)) // NB, (kk % (2 * NB)) % NB],
                out_hbm.at[cid, pl.ds(r0 + kk * CK, CK)],
                gsem.at[(kk % (2 * NB)) // NB, (kk % (2 * NB)) % NB]).wait()

        if REM:
            pltpu.sync_copy(acc.at[pl.ds(r0 + NFULL * CK, REM)],
                            rows_v.at[0, 0, pl.ds(0, REM)])
            pltpu.sync_copy(rows_v.at[0, 0, pl.ds(0, REM)],
                            out_hbm.at[cid, pl.ds(r0 + NFULL * CK, REM)])

    return k(h, srcg, dstg)


def _tc_mlp(h, p0, p1, Wi, bi, gi, bei, Wo, bo, final_relu):
    def body(h_ref, p0_ref, p1_ref, wi_ref, bi_ref, gi_ref, bei_ref,
             wo_ref, bo_ref, o_ref):
        z = h_ref[...] + p0_ref[...] + p1_ref[...]
        z = jnp.dot(z, wi_ref[...], preferred_element_type=jnp.float32)
        z = z + bi_ref[...]
        mu = jnp.mean(z, axis=0, keepdims=True)
        var = jnp.mean((z - mu) ** 2, axis=0, keepdims=True)
        z = (z - mu) / jnp.sqrt(var + BN_EPS) * gi_ref[...] + bei_ref[...]
        z = jnp.maximum(z, 0.0)
        z = jnp.dot(z, wo_ref[...], preferred_element_type=jnp.float32)
        z = z + bo_ref[...]
        if final_relu:
            z = jnp.maximum(z, 0.0)
        o_ref[...] = z

    return pl.pallas_call(
        body,
        out_shape=jax.ShapeDtypeStruct((N, D), jnp.float32),
    )(h, p0, p1, Wi, bi.reshape(1, D), gi.reshape(1, D), bei.reshape(1, D),
      Wo, bo.reshape(1, D))


def kernel(x, edge_index, W1a, b1a, g1a, be1a, W2a, b2a,
           W1b, b1b, g1b, be1b, W2b, b2b):
    srcg = edge_index[0].reshape(NW, NGP, NB, CK)
    dstg = edge_index[1].reshape(NW, NGP, NB, CK)

    p = _sc_agg(x, srcg, dstg)
    h1 = _tc_mlp(x, p[0, :N], p[1, :N], W1a, b1a, g1a, be1a, W2a, b2a,
                 final_relu=True)
    p2 = _sc_agg(h1, srcg, dstg)
    out = _tc_mlp(h1, p2[0, :N], p2[1, :N], W1b, b1b, g1b, be1b, W2b, b2b,
                  final_relu=False)
    return out


# speedup vs baseline: 1.0854x; 1.0854x over previous
import functools

import jax
import jax.numpy as jnp
from jax import lax
from jax.experimental import pallas as pl
from jax.experimental.pallas import tpu as pltpu
from jax.experimental.pallas import tpu_sc as plsc

N = 10000
E = 320000
D = 128
BN_EPS = 1e-5

NC = 2
NS = 16
NW = NC * NS
EPT = E // NW
CK = 40
CH = EPT // CK
NP = 10112
RPT = NP // NS
NB = 2
NGP = CH // NB
NSL = 4
NFULL = RPT // CK
REM = RPT - NFULL * CK


def _sc_agg(h, srcg, dstg):
    mesh = plsc.VectorSubcoreMesh(core_axis_name="c", subcore_axis_name="s")

    @functools.partial(
        pl.kernel,
        mesh=mesh,
        out_type=jax.ShapeDtypeStruct((NC, NP, D), jnp.float32),
        scratch_types=[
            pltpu.VMEM((NSL, NB, CK), jnp.int32),
            pltpu.VMEM((NSL, NB, CK), jnp.int32),
            pltpu.VMEM((2, NB, CK, D), jnp.float32),
            pltpu.VMEM_SHARED((NP, D), jnp.float32),
            pltpu.SemaphoreType.DMA((2, NB)),
            pltpu.SemaphoreType.DMA((2, NB)),
            pltpu.SemaphoreType.DMA((NSL,)),
            pltpu.SemaphoreType.DMA((NSL,)),
        ],
    )
    def k(h_hbm, srcg_hbm, dstg_hbm, out_hbm,
          sidx, didx, rows_v, acc, gsem, ssem, isems, isemd):
        cid = lax.axis_index("c")
        sid = lax.axis_index("s")
        wid = cid * NS + sid
        r0 = sid * RPT

        pltpu.async_copy(srcg_hbm.at[wid, 0], sidx.at[0], isems.at[0])
        pltpu.async_copy(dstg_hbm.at[wid, 0], didx.at[0], isemd.at[0])
        pltpu.async_copy(srcg_hbm.at[wid, 1], sidx.at[1], isems.at[1])
        pltpu.async_copy(dstg_hbm.at[wid, 1], didx.at[1], isemd.at[1])

        zv = jnp.zeros((16,), jnp.float32)

        @pl.loop(0, CK)
        def _(i):
            for q in range(D // 16):
                rows_v[0, 0, i, pl.ds(q * 16, 16)] = zv

        @pl.loop(0, NFULL)
        def _(kk):
            pltpu.async_copy(rows_v.at[0, 0],
                             acc.at[pl.ds(r0 + kk * CK, CK)], gsem.at[0, 0])

        @pl.loop(0, NFULL)
        def _(kk):
            pltpu.make_async_copy(
                rows_v.at[0, 0], acc.at[pl.ds(r0 + kk * CK, CK)],
                gsem.at[0, 0]).wait()

        if REM:
            pltpu.sync_copy(rows_v.at[0, 0, pl.ds(0, REM)],
                            acc.at[pl.ds(r0 + NFULL * CK, REM)])

        pltpu.make_async_copy(srcg_hbm.at[wid, 0], sidx.at[0],
                              isems.at[0]).wait()
        pltpu.make_async_copy(dstg_hbm.at[wid, 0], didx.at[0],
                              isemd.at[0]).wait()
        plsc.subcore_barrier()
        for b in range(NB):
            pltpu.async_copy(h_hbm.at[sidx.at[0, b]], rows_v.at[0, b],
                             gsem.at[0, b])

        @pl.loop(0, NGP)
        def _(g):
            w = lax.rem(g, 2)
            s = lax.rem(g, NSL)

            @pl.when(g + 2 < NGP)
            def _():
                s2 = lax.rem(g + 2, NSL)
                pltpu.async_copy(srcg_hbm.at[wid, g + 2], sidx.at[s2],
                                 isems.at[s2])
                pltpu.async_copy(dstg_hbm.at[wid, g + 2], didx.at[s2],
                                 isemd.at[s2])

            for b in range(NB):
                pltpu.make_async_copy(
                    h_hbm.at[sidx.at[s, b]], rows_v.at[w, b], gsem.at[w, b]
                ).wait()
                pltpu.async_copy(
                    rows_v.at[w, b], acc.at[didx.at[s, b]], ssem.at[w, b],
                    add=True)

            @pl.when(g > 0)
            def _():
                wp = lax.rem(g + 1, 2)
                sp = lax.rem(g + NSL - 1, NSL)
                for b in range(NB):
                    pltpu.make_async_copy(
                        rows_v.at[wp, b], acc.at[didx.at[sp, b]],
                        ssem.at[wp, b]).wait()

            @pl.when(g + 1 < NGP)
            def _():
                wn = lax.rem(g + 1, 2)
                sn = lax.rem(g + 1, NSL)
                pltpu.make_async_copy(srcg_hbm.at[wid, g + 1], sidx.at[sn],
                                      isems.at[sn]).wait()
                pltpu.make_async_copy(dstg_hbm.at[wid, g + 1], didx.at[sn],
                                      isemd.at[sn]).wait()
                for b in range(NB):
                    pltpu.async_copy(h_hbm.at[sidx.at[sn, b]],
                                     rows_v.at[wn, b], gsem.at[wn, b])

        wl = (NGP - 1) % 2
        sl = (NGP - 1) % NSL
        for b in range(NB):
            pltpu.make_async_copy(rows_v.at[wl, b], acc.at[didx.at[sl, b]],
                                  ssem.at[wl, b]).wait()

        plsc.subcore_barrier()

        for q in range(2 * NB):
            pltpu.async_copy(acc.at[pl.ds(r0 + q * CK, CK)],
                             rows_v.at[q // NB, q % NB],
                             ssem.at[q // NB, q % NB])

        @pl.loop(0, NFULL)
        def _(kk):
            q = lax.rem(kk, 2 * NB)
            qw = q // NB
            qb = lax.rem(q, NB)
            pltpu.make_async_copy(
                acc.at[pl.ds(r0 + kk * CK, CK)], rows_v.at[qw, qb],
                ssem.at[qw, qb]).wait()
            pltpu.async_copy(rows_v.at[qw, qb],
                             out_hbm.at[cid, pl.ds(r0 + kk * CK, CK)],
                             gsem.at[qw, qb])

            @pl.when(kk + 2 * NB < NFULL)
            def _():
                pltpu.make_async_copy(
                    rows_v.at[qw, qb],
                    out_hbm.at[cid, pl.ds(r0 + kk * CK, CK)],
                    gsem.at[qw, qb]).wait()
                pltpu.async_copy(
                    acc.at[pl.ds(r0 + (kk + 2 * NB) * CK, CK)],
                    rows_v.at[qw, qb], ssem.at[qw, qb])

        for q in range(2 * NB):
            kk = NFULL - 2 * NB + q
            pltpu.make_async_copy(
                rows_v.at[(kk % (2 * NB)) // NB, (kk % (2 * NB)) % NB],
                out_hbm.at[cid, pl.ds(r0 + kk * CK, CK)],
                gsem.at[(kk % (2 * NB)) // NB, (kk % (2 * NB)) % NB]).wait()

        if REM:
            pltpu.sync_copy(acc.at[pl.ds(r0 + NFULL * CK, REM)],
                            rows_v.at[0, 0, pl.ds(0, REM)])
            pltpu.sync_copy(rows_v.at[0, 0, pl.ds(0, REM)],
                            out_hbm.at[cid, pl.ds(r0 + NFULL * CK, REM)])

    return k(h, srcg, dstg)


def _tc_mlp(h, p, Wi, bi, gi, bei, Wo, bo, final_relu):
    def body(h_ref, p_ref, wi_ref, bi_ref, gi_ref, bei_ref,
             wo_ref, bo_ref, o_ref):
        z = h_ref[...] + p_ref[0, :N, :] + p_ref[1, :N, :]
        z = jnp.dot(z, wi_ref[...], preferred_element_type=jnp.float32)
        z = z + bi_ref[...]
        mu = jnp.mean(z, axis=0, keepdims=True)
        var = jnp.mean((z - mu) ** 2, axis=0, keepdims=True)
        z = (z - mu) / jnp.sqrt(var + BN_EPS) * gi_ref[...] + bei_ref[...]
        z = jnp.maximum(z, 0.0)
        z = jnp.dot(z, wo_ref[...], preferred_element_type=jnp.float32)
        z = z + bo_ref[...]
        if final_relu:
            z = jnp.maximum(z, 0.0)
        o_ref[...] = z

    return pl.pallas_call(
        body,
        out_shape=jax.ShapeDtypeStruct((N, D), jnp.float32),
    )(h, p, Wi, bi.reshape(1, D), gi.reshape(1, D), bei.reshape(1, D),
      Wo, bo.reshape(1, D))


def kernel(x, edge_index, W1a, b1a, g1a, be1a, W2a, b2a,
           W1b, b1b, g1b, be1b, W2b, b2b):
    srcg = edge_index[0].reshape(NW, NGP, NB, CK)
    dstg = edge_index[1].reshape(NW, NGP, NB, CK)

    p = _sc_agg(x, srcg, dstg)
    h1 = _tc_mlp(x, p, W1a, b1a, g1a, be1a, W2a, b2a, final_relu=True)
    p2 = _sc_agg(h1, srcg, dstg)
    out = _tc_mlp(h1, p2, W1b, b1b, g1b, be1b, W2b, b2b, final_relu=False)
    return out
